# baseline (speedup 1.0000x reference)
import functools

import jax
import jax.numpy as jnp
from jax import lax
from jax.experimental import pallas as pl
from jax.experimental.pallas import tpu as pltpu
from jax.experimental.pallas import tpu_sc as plsc

VOCAB_SIZE = 100000
N_TOKENS = 512
NUM_CORES = 2
NUM_SUBCORES = 16
NUM_WORKERS = NUM_CORES * NUM_SUBCORES
ROWS_PER_WORKER = N_TOKENS // NUM_WORKERS
HALF = VOCAB_SIZE // 2
QUART = VOCAB_SIZE // 4
QUART_PAD = QUART + 16

_mesh = plsc.VectorSubcoreMesh(
    core_axis_name="c",
    subcore_axis_name="s",
    num_cores=NUM_CORES,
    num_subcores=NUM_SUBCORES,
)


@functools.partial(
    pl.kernel,
    out_type=jax.ShapeDtypeStruct((N_TOKENS * VOCAB_SIZE,), jnp.float32),
    mesh=_mesh,
    scratch_types=[
        pltpu.VMEM((ROWS_PER_WORKER,), jnp.int32),
        pltpu.VMEM((HALF,), jnp.float32),
        pltpu.VMEM((QUART_PAD,), jnp.float32),
        pltpu.VMEM((QUART_PAD,), jnp.float32),
        pltpu.SemaphoreType.DMA,
        pltpu.SemaphoreType.DMA,
        pltpu.SemaphoreType.DMA,
    ],
)
def _sc_onehot(ids_hbm, zero_hbm, out_hbm, ids_v, zb, h0, h1,
               sem_z, sem_h0, sem_h1):
    wid = lax.axis_index("s") * NUM_CORES + lax.axis_index("c")
    base = wid * ROWS_PER_WORKER

    pltpu.sync_copy(ids_hbm.at[pl.ds(base, ROWS_PER_WORKER)], ids_v)
    pltpu.sync_copy(zero_hbm, zb)
    pltpu.sync_copy(zero_hbm.at[pl.ds(0, QUART_PAD)], h0)
    pltpu.sync_copy(zero_hbm.at[pl.ds(0, QUART_PAD)], h1)

    lane = lax.iota(jnp.int32, 16)
    vals_zero = jnp.zeros((16,), jnp.float32)

    idx_vec = (ids_v[...] + 1) % VOCAB_SIZE

    hbufs = [h0, h1]
    hsems = [sem_h0, sem_h1]
    hot_desc = [None, None]
    hot_prev_off = [None, None]
    zero_descs = []

    for t in range(ROWS_PER_WORKER):
        idx_t = idx_vec[t]
        q = idx_t // QUART
        half_hot = q // 2
        off_in = idx_t - q * QUART
        row_off = (base + t) * VOCAB_SIZE

        d1 = pltpu.make_async_copy(
            zb, out_hbm.at[pl.ds(row_off + (1 - half_hot) * HALF, HALF)],
            sem_z)
        d1.start()
        sib = half_hot * HALF + (1 - (q - half_hot * 2)) * QUART
        d2 = pltpu.make_async_copy(
            zb.at[pl.ds(0, QUART)], out_hbm.at[pl.ds(row_off + sib, QUART)],
            sem_z)
        d2.start()
        zero_descs += [d1, d2]

        hb = t % 2
        off_base = (off_in // 16) * 16
        lane_pos = off_in - off_base
        if hot_desc[hb] is not None:
            hot_desc[hb].wait()
            hbufs[hb][pl.ds(hot_prev_off[hb], 16)] = vals_zero
        hbufs[hb][pl.ds(off_base, 16)] = jnp.where(
            lane == lane_pos, jnp.float32(20.0), jnp.float32(0.0))
        hd = pltpu.make_async_copy(
            hbufs[hb].at[pl.ds(0, QUART)],
            out_hbm.at[pl.ds(row_off + q * QUART, QUART)],
            hsems[hb])
        hd.start()
        hot_desc[hb] = hd
        hot_prev_off[hb] = off_base

    for d in zero_descs:
        d.wait()
    hot_desc[0].wait()
    hot_desc[1].wait()


def kernel(input_ids):
    B, S = input_ids.shape
    ids = input_ids.reshape(-1).astype(jnp.int32)
    zero_tile = jnp.zeros((HALF,), jnp.float32)
    out = _sc_onehot(ids, zero_tile)
    return out.reshape(B, S, VOCAB_SIZE)


# speedup vs baseline: 4.6553x; 4.6553x over previous
import functools

import jax
import jax.numpy as jnp
from jax import lax
from jax.experimental import pallas as pl
from jax.experimental.pallas import tpu as pltpu
from jax.experimental.pallas import tpu_sc as plsc

VOCAB_SIZE = 100000
N_TOKENS = 512
LANE = 128
EDGE_START = (VOCAB_SIZE // LANE) * LANE
NUM_CORES = 2
NUM_SUBCORES = 16
NUM_WORKERS = NUM_CORES * NUM_SUBCORES
ROWS_PER_WORKER = N_TOKENS // NUM_WORKERS
BV = 2048


def _tc_fill_kernel(idx_ref, out_ref):
    j = pl.program_id(0)
    col0 = j * BV
    rows, bv = out_ref.shape
    idx = idx_ref[...]
    idx_eff = jnp.where(idx >= EDGE_START, idx, -1)
    iota = jax.lax.broadcasted_iota(jnp.int32, (rows, bv), 1) + col0
    out_ref[...] = jnp.where(iota == idx_eff, 20.0, 0.0).astype(jnp.float32)


def _tc_fill(idx):
    n = idx.shape[0]
    grid = pl.cdiv(VOCAB_SIZE, BV)
    return pl.pallas_call(
        _tc_fill_kernel,
        grid=(grid,),
        in_specs=[pl.BlockSpec((n, 1), lambda j: (0, 0))],
        out_specs=pl.BlockSpec((n, BV), lambda j: (0, j)),
        out_shape=jax.ShapeDtypeStruct((n, VOCAB_SIZE), jnp.float32),
    )(idx.reshape(n, 1))


_mesh = plsc.VectorSubcoreMesh(
    core_axis_name="c",
    subcore_axis_name="s",
    num_cores=NUM_CORES,
    num_subcores=NUM_SUBCORES,
)


@functools.partial(
    pl.kernel,
    mesh=_mesh,
    scratch_types=[
        pltpu.VMEM((ROWS_PER_WORKER,), jnp.int32),
        pltpu.VMEM((8, LANE), jnp.float32),
        pltpu.VMEM((8, LANE), jnp.float32),
        pltpu.SemaphoreType.DMA,
        pltpu.SemaphoreType.DMA,
    ],
)
def _sc_patch(ids_hbm, out_ref, ids_v, pb0, pb1, sem0, sem1):
    wid = lax.axis_index("s") * NUM_CORES + lax.axis_index("c")
    base = wid * ROWS_PER_WORKER

    pltpu.sync_copy(ids_hbm.at[pl.ds(base, ROWS_PER_WORKER)], ids_v)
    idx_vec = (ids_v[...] + 1) % VOCAB_SIZE

    lane = lax.iota(jnp.int32, 16)
    zeros16 = jnp.zeros((16,), jnp.float32)

    pbs = [pb0, pb1]
    sems = [sem0, sem1]
    last_desc = [None, None]

    for g in range(2):
        rows0 = base + g * 8

        for hb in range(2):
            if last_desc[hb] is not None:
                last_desc[hb].wait()
                last_desc[hb] = None
            for rr in range(8):
                for k in range(LANE // 16):
                    pbs[hb][rr, pl.ds(k * 16, 16)] = zeros16

        idx_s = [idx_vec[g * 8 + r2] for r2 in range(8)]
        ct_s = [ix // LANE for ix in idx_s]
        edge_s = [ix >= EDGE_START for ix in idx_s]
        b16_s = [((ix % LANE) // 16) * 16 for ix in idx_s]
        o16_s = [ix % 16 for ix in idx_s]
        ct_eff_s = [jnp.where(edge_s[r], 0, ct_s[r]) for r in range(8)]

        for r in range(8):
            hb = r % 2
            if last_desc[hb] is not None:
                last_desc[hb].wait()
            ct = ct_eff_s[r]
            for r2 in range(8):
                contrib = jnp.where(
                    (ct_s[r2] == ct) & jnp.logical_not(edge_s[r2]),
                    jnp.float32(20.0), jnp.float32(0.0))
                pbs[hb][r2, pl.ds(b16_s[r2], 16)] = jnp.where(
                    lane == o16_s[r2], contrib, jnp.float32(0.0))
            d = pltpu.make_async_copy(
                pbs[hb],
                out_ref.at[pl.ds(rows0, 8), pl.ds(ct * LANE, LANE)],
                sems[hb])
            d.start()
            last_desc[hb] = d

    last_desc[0].wait()
    last_desc[1].wait()


def kernel(input_ids):
    B, S = input_ids.shape
    ids = input_ids.reshape(-1).astype(jnp.int32)
    idx = (ids + 1) % VOCAB_SIZE
    dense = _tc_fill(idx)
    ref = jax.new_ref(dense)
    _sc_patch(ids, ref)
    return ref[...].reshape(B, S, VOCAB_SIZE)


# speedup vs baseline: 4.7264x; 1.0153x over previous
import functools

import jax
import jax.numpy as jnp
from jax import lax
from jax.experimental import pallas as pl
from jax.experimental.pallas import tpu as pltpu
from jax.experimental.pallas import tpu_sc as plsc

VOCAB_SIZE = 100000
N_TOKENS = 512
LANE = 128
EDGE_START = (VOCAB_SIZE // LANE) * LANE
NUM_CORES = 2
NUM_SUBCORES = 16
NUM_WORKERS = NUM_CORES * NUM_SUBCORES
ROWS_PER_WORKER = N_TOKENS // NUM_WORKERS
BV = 2048


def _tc_fill_kernel(ids_ref, out_ref):
    j = pl.program_id(0)
    col0 = j * BV
    rows, bv = out_ref.shape
    idx = (ids_ref[...] + 1) % VOCAB_SIZE
    idx_eff = jnp.where(idx >= EDGE_START, idx, -1)
    iota = jax.lax.broadcasted_iota(jnp.int32, (rows, bv), 1) + col0
    out_ref[...] = jnp.where(iota == idx_eff, 20.0, 0.0).astype(jnp.float32)


def _tc_fill(ids):
    n = ids.shape[0]
    grid = pl.cdiv(VOCAB_SIZE, BV)
    return pl.pallas_call(
        _tc_fill_kernel,
        grid=(grid,),
        in_specs=[pl.BlockSpec((n, 1), lambda j: (0, 0))],
        out_specs=pl.BlockSpec((n, BV), lambda j: (0, j)),
        out_shape=jax.ShapeDtypeStruct((n, VOCAB_SIZE), jnp.float32),
    )(ids.reshape(n, 1))


_mesh = plsc.VectorSubcoreMesh(
    core_axis_name="c",
    subcore_axis_name="s",
    num_cores=NUM_CORES,
    num_subcores=NUM_SUBCORES,
)


@functools.partial(
    pl.kernel,
    mesh=_mesh,
    scratch_types=[
        pltpu.VMEM((ROWS_PER_WORKER,), jnp.int32),
        pltpu.VMEM((8, LANE), jnp.float32),
        pltpu.VMEM((8, LANE), jnp.float32),
        pltpu.SemaphoreType.DMA,
        pltpu.SemaphoreType.DMA,
    ],
)
def _sc_patch(ids_hbm, out_ref, ids_v, pb0, pb1, sem0, sem1):
    wid = lax.axis_index("s") * NUM_CORES + lax.axis_index("c")
    base = wid * ROWS_PER_WORKER

    pltpu.sync_copy(ids_hbm.at[pl.ds(base, ROWS_PER_WORKER)], ids_v)
    idx_vec = (ids_v[...] + 1) % VOCAB_SIZE

    lane = lax.iota(jnp.int32, 16)
    zeros16 = jnp.zeros((16,), jnp.float32)

    pbs = [pb0, pb1]
    sems = [sem0, sem1]
    last_desc = [None, None]

    for g in range(2):
        rows0 = base + g * 8

        for hb in range(2):
            if last_desc[hb] is not None:
                last_desc[hb].wait()
                last_desc[hb] = None
            for rr in range(8):
                for k in range(LANE // 16):
                    pbs[hb][rr, pl.ds(k * 16, 16)] = zeros16

        idx_s = [idx_vec[g * 8 + r2] for r2 in range(8)]
        ct_s = [ix // LANE for ix in idx_s]
        edge_s = [ix >= EDGE_START for ix in idx_s]
        b16_s = [((ix % LANE) // 16) * 16 for ix in idx_s]
        o16_s = [ix % 16 for ix in idx_s]
        ct_eff_s = [jnp.where(edge_s[r], 0, ct_s[r]) for r in range(8)]

        for r in range(8):
            hb = r % 2
            if last_desc[hb] is not None:
                last_desc[hb].wait()
            ct = ct_eff_s[r]
            for r2 in range(8):
                contrib = jnp.where(
                    (ct_s[r2] == ct) & jnp.logical_not(edge_s[r2]),
                    jnp.float32(20.0), jnp.float32(0.0))
                pbs[hb][r2, pl.ds(b16_s[r2], 16)] = jnp.where(
                    lane == o16_s[r2], contrib, jnp.float32(0.0))
            d = pltpu.make_async_copy(
                pbs[hb],
                out_ref.at[pl.ds(rows0, 8), pl.ds(ct * LANE, LANE)],
                sems[hb])
            d.start()
            last_desc[hb] = d

    last_desc[0].wait()
    last_desc[1].wait()


def kernel(input_ids):
    B, S = input_ids.shape
    ids = input_ids.reshape(-1).astype(jnp.int32)
    dense = _tc_fill(ids)
    ref = jax.new_ref(dense)
    _sc_patch(ids, ref)
    return ref[...].reshape(B, S, VOCAB_SIZE)
